# baseline (device time: 31562 ns/iter reference)
import os

import jax
import jax.numpy as jnp
from jax import lax
from jax.experimental import pallas as pl
from jax.experimental.pallas import tpu as pltpu

N_DEV = 4
BLOCK = 64
DH = 64
B = 2
SQ = 512
DM = 768
HALF = DM // 2
H = 8
NCHUNK = 4
R = B * SQ // NCHUNK

_COMM = os.environ.get("KERNEL_NO_COMM") != "1"


def _fused_body(
    x_ref, wq_ref, k_ref, v_ref, wo_ref, out_ref,
    pa, pb, acc_a, acc_b, ra1, rb1, ra2, rb2,
    send_sems, recv_sems,
):
    my = lax.axis_index("i")
    p_y = my ^ 1
    p_x = 3 - my
    bf = jnp.bfloat16
    f32 = jnp.float32

    barrier = pltpu.get_barrier_semaphore()
    for nbr in [p_y, p_x]:
        pl.semaphore_signal(
            barrier, inc=1, device_id=(nbr,), device_id_type=pl.DeviceIdType.MESH
        )

    qi = lax.broadcasted_iota(jnp.int32, (SQ, SQ), 0) // BLOCK
    kj = lax.broadcasted_iota(jnp.int32, (SQ, SQ), 1) // BLOCK
    mask = (qi == kj) | (kj == 0) | ((qi + kj) % 3 == 0)
    bias = jnp.where(mask, 0.0, -1e9).astype(f32)

    wqv = wq_ref[...].astype(bf)
    wov = wo_ref[...].astype(bf)

    def compute_chunk(c):
        r0 = c * R
        b, o = r0 // SQ, r0 % SQ
        q = jnp.dot(
            x_ref[b, o:o + R, :].astype(bf), wqv, preferred_element_type=f32
        )
        bias_c = bias[o:o + R, :]
        ctx_heads = []
        for h in range(H):
            qh = q[:, h * DH:(h + 1) * DH].astype(bf)
            kht = k_ref[b, h]
            s = jnp.dot(qh, kht, preferred_element_type=f32) * 0.125 + bias_c
            e = jnp.exp(s)
            w = (e * (1.0 / e.sum(axis=-1, keepdims=True))).astype(bf)
            ctx_heads.append(
                jnp.dot(w, v_ref[b, h], preferred_element_type=f32)
            )
        ctx = jnp.concatenate(ctx_heads, axis=1).astype(bf)
        part = jnp.dot(ctx, wov, preferred_element_type=f32)
        pa[r0:r0 + R, :] = part[:, :HALF].astype(bf)
        pb[r0:r0 + R, :] = part[:, HALF:].astype(bf)

    def phase1(c):
        r0 = c * R
        ca = pltpu.make_async_remote_copy(
            src_ref=pa.at[r0:r0 + R, :], dst_ref=ra1.at[r0:r0 + R, :],
            send_sem=send_sems.at[4 * c + 0], recv_sem=recv_sems.at[4 * c + 0],
            device_id=(p_y,), device_id_type=pl.DeviceIdType.MESH,
        )
        cb = pltpu.make_async_remote_copy(
            src_ref=pb.at[r0:r0 + R, :], dst_ref=rb1.at[r0:r0 + R, :],
            send_sem=send_sems.at[4 * c + 1], recv_sem=recv_sems.at[4 * c + 1],
            device_id=(p_x,), device_id_type=pl.DeviceIdType.MESH,
        )
        ca.start()
        cb.start()
        return ca, cb

    def phase2(c):
        r0 = c * R
        ca = pltpu.make_async_remote_copy(
            src_ref=acc_a.at[r0:r0 + R, :], dst_ref=ra2.at[r0:r0 + R, :],
            send_sem=send_sems.at[4 * c + 2], recv_sem=recv_sems.at[4 * c + 2],
            device_id=(p_x,), device_id_type=pl.DeviceIdType.MESH,
        )
        cb = pltpu.make_async_remote_copy(
            src_ref=acc_b.at[r0:r0 + R, :], dst_ref=rb2.at[r0:r0 + R, :],
            send_sem=send_sems.at[4 * c + 3], recv_sem=recv_sems.at[4 * c + 3],
            device_id=(p_y,), device_id_type=pl.DeviceIdType.MESH,
        )
        ca.start()
        cb.start()
        return ca, cb

    def recv2(pair):
        pair[0].wait_recv()
        pair[1].wait_recv()

    def add1(c):
        sl = pl.ds(c * R, R)
        acc_a[sl, :] = pa[sl, :] + ra1[sl, :]
        acc_b[sl, :] = pb[sl, :] + rb1[sl, :]

    def emit(c):
        sl = pl.ds(c * R, R)
        out_ref[sl, :HALF] = (
            acc_a[sl, :].astype(f32) + ra2[sl, :].astype(f32)
        ).astype(jnp.bfloat16)
        out_ref[sl, HALF:] = (
            acc_b[sl, :].astype(f32) + rb2[sl, :].astype(f32)
        ).astype(jnp.bfloat16)

    if not _COMM:
        for c in range(NCHUNK):
            compute_chunk(c)
        for c in range(NCHUNK):
            sl = pl.ds(c * R, R)
            out_ref[sl, :HALF] = pa[sl, :]
            out_ref[sl, HALF:] = pb[sl, :]
        return

    p1 = [None] * NCHUNK
    p2 = [None] * NCHUNK
    for c in range(NCHUNK):
        compute_chunk(c)
        if c == 0:
            pl.semaphore_wait(barrier, 2)
        p1[c] = phase1(c)
        if c >= 1:
            recv2(p1[c - 1]); add1(c - 1); p2[c - 1] = phase2(c - 1)
    recv2(p1[NCHUNK - 1]); add1(NCHUNK - 1); p2[NCHUNK - 1] = phase2(NCHUNK - 1)
    for c in range(NCHUNK):
        recv2(p2[c]); emit(c)
    for c in range(NCHUNK):
        p1[c][0].wait_send(); p1[c][1].wait_send()
        p2[c][0].wait_send(); p2[c][1].wait_send()


def kernel(x, Wq, K_ext, V_ext, Wo):
    bf = jnp.bfloat16
    my = lax.axis_index("i")
    K = lax.dynamic_slice_in_dim(K_ext, my * H, H, axis=2)
    V = lax.dynamic_slice_in_dim(V_ext, my * H, H, axis=2)
    Kt = jnp.transpose(K, (0, 2, 3, 1)).astype(bf)
    Vt = jnp.transpose(V, (0, 2, 1, 3)).astype(bf)

    m = B * SQ
    half_buf = pltpu.VMEM((m, HALF), bf)
    out = pl.pallas_call(
        _fused_body,
        out_shape=jax.ShapeDtypeStruct((m, DM), jnp.bfloat16),
        in_specs=[pl.BlockSpec(memory_space=pltpu.VMEM)] * 5,
        out_specs=pl.BlockSpec(memory_space=pltpu.VMEM),
        scratch_shapes=[
            half_buf, half_buf,
            half_buf, half_buf,
            half_buf, half_buf, half_buf, half_buf,
            pltpu.SemaphoreType.DMA((4 * NCHUNK,)),
            pltpu.SemaphoreType.DMA((4 * NCHUNK,)),
        ],
        compiler_params=pltpu.CompilerParams(collective_id=0),
    )(x, Wq, Kt, Vt, Wo)
    return out.reshape(B, SQ, DM)


# device time: 14745 ns/iter; 2.1405x vs baseline; 2.1405x over previous
import os

import jax
import jax.numpy as jnp
from jax import lax
from jax.experimental import pallas as pl
from jax.experimental.pallas import tpu as pltpu

N_DEV = 4
BLOCK = 64
DH = 64
B = 2
SQ = 512
DM = 768
HALF = DM // 2
H = 8
NCHUNK = 4
R = B * SQ // NCHUNK

_COMM = os.environ.get("KERNEL_NO_COMM") != "1"


def _fused_body(
    x_ref, wq_ref, k_ref, v_ref, wo_ref, out_ref,
    pa, pb, acc_a, acc_b, ra1, rb1, ra2, rb2,
    send_sems, recv_sems,
):
    my = lax.axis_index("i")
    p_y = my ^ 1
    p_x = 3 - my
    bf = jnp.bfloat16
    f32 = jnp.float32

    barrier = pltpu.get_barrier_semaphore()
    for nbr in [p_y, p_x]:
        pl.semaphore_signal(
            barrier, inc=1, device_id=(nbr,), device_id_type=pl.DeviceIdType.MESH
        )

    wqv = wq_ref[...].astype(bf)
    wov = wo_ref[...].astype(bf)

    def kept_blocks(qb):
        ks = {0, qb}
        for kb in range(SQ // BLOCK):
            if (qb + kb) % 3 == 0:
                ks.add(kb)
        return sorted(ks)

    def compute_chunk(c):
        r0 = c * R
        b, o = r0 // SQ, r0 % SQ
        q = jnp.dot(
            x_ref[b, o:o + R, :].astype(bf), wqv, preferred_element_type=f32
        )
        ctx_rows = []
        for j in range(R // BLOCK):
            qb = o // BLOCK + j
            kept = kept_blocks(qb)
            qs = q[j * BLOCK:(j + 1) * BLOCK, :]
            q3 = jnp.stack(
                [qs[:, h * DH:(h + 1) * DH] for h in range(H)], axis=0
            ).astype(bf)
            k_sel = jnp.concatenate(
                [k_ref[b, :, :, kb * BLOCK:(kb + 1) * BLOCK] for kb in kept],
                axis=2,
            )
            s3 = lax.dot_general(
                q3, k_sel, (((2,), (1,)), ((0,), (0,))),
                preferred_element_type=f32,
            ) * 0.125
            e = jnp.exp(s3)
            w = (e * (1.0 / e.sum(axis=-1, keepdims=True))).astype(bf)
            v_sel = jnp.concatenate(
                [v_ref[b, :, kb * BLOCK:(kb + 1) * BLOCK, :] for kb in kept],
                axis=1,
            )
            c3 = lax.dot_general(
                w, v_sel, (((2,), (1,)), ((0,), (0,))),
                preferred_element_type=f32,
            )
            ctx_rows.append(
                jnp.concatenate([c3[h] for h in range(H)], axis=1)
            )
        ctx = jnp.concatenate(ctx_rows, axis=0).astype(bf)
        part = jnp.dot(ctx, wov, preferred_element_type=f32)
        pa[r0:r0 + R, :] = part[:, :HALF].astype(bf)
        pb[r0:r0 + R, :] = part[:, HALF:].astype(bf)

    def phase1(c):
        r0 = c * R
        ca = pltpu.make_async_remote_copy(
            src_ref=pa.at[r0:r0 + R, :], dst_ref=ra1.at[r0:r0 + R, :],
            send_sem=send_sems.at[4 * c + 0], recv_sem=recv_sems.at[4 * c + 0],
            device_id=(p_y,), device_id_type=pl.DeviceIdType.MESH,
        )
        cb = pltpu.make_async_remote_copy(
            src_ref=pb.at[r0:r0 + R, :], dst_ref=rb1.at[r0:r0 + R, :],
            send_sem=send_sems.at[4 * c + 1], recv_sem=recv_sems.at[4 * c + 1],
            device_id=(p_x,), device_id_type=pl.DeviceIdType.MESH,
        )
        ca.start()
        cb.start()
        return ca, cb

    def phase2(c):
        r0 = c * R
        ca = pltpu.make_async_remote_copy(
            src_ref=acc_a.at[r0:r0 + R, :], dst_ref=ra2.at[r0:r0 + R, :],
            send_sem=send_sems.at[4 * c + 2], recv_sem=recv_sems.at[4 * c + 2],
            device_id=(p_x,), device_id_type=pl.DeviceIdType.MESH,
        )
        cb = pltpu.make_async_remote_copy(
            src_ref=acc_b.at[r0:r0 + R, :], dst_ref=rb2.at[r0:r0 + R, :],
            send_sem=send_sems.at[4 * c + 3], recv_sem=recv_sems.at[4 * c + 3],
            device_id=(p_y,), device_id_type=pl.DeviceIdType.MESH,
        )
        ca.start()
        cb.start()
        return ca, cb

    def recv2(pair):
        pair[0].wait_recv()
        pair[1].wait_recv()

    def add1(c):
        sl = pl.ds(c * R, R)
        acc_a[sl, :] = pa[sl, :] + ra1[sl, :]
        acc_b[sl, :] = pb[sl, :] + rb1[sl, :]

    def emit(c):
        sl = pl.ds(c * R, R)
        out_ref[sl, :HALF] = (
            acc_a[sl, :].astype(f32) + ra2[sl, :].astype(f32)
        ).astype(jnp.bfloat16)
        out_ref[sl, HALF:] = (
            acc_b[sl, :].astype(f32) + rb2[sl, :].astype(f32)
        ).astype(jnp.bfloat16)

    if not _COMM:
        for c in range(NCHUNK):
            compute_chunk(c)
        for c in range(NCHUNK):
            sl = pl.ds(c * R, R)
            out_ref[sl, :HALF] = pa[sl, :]
            out_ref[sl, HALF:] = pb[sl, :]
        return

    p1 = [None] * NCHUNK
    p2 = [None] * NCHUNK
    for c in range(NCHUNK):
        compute_chunk(c)
        if c == 0:
            pl.semaphore_wait(barrier, 2)
        p1[c] = phase1(c)
        if c >= 1:
            recv2(p1[c - 1]); add1(c - 1); p2[c - 1] = phase2(c - 1)
    recv2(p1[NCHUNK - 1]); add1(NCHUNK - 1); p2[NCHUNK - 1] = phase2(NCHUNK - 1)
    for c in range(NCHUNK):
        recv2(p2[c]); emit(c)
    for c in range(NCHUNK):
        p1[c][0].wait_send(); p1[c][1].wait_send()
        p2[c][0].wait_send(); p2[c][1].wait_send()


def kernel(x, Wq, K_ext, V_ext, Wo):
    bf = jnp.bfloat16
    my = lax.axis_index("i")
    K = lax.dynamic_slice_in_dim(K_ext, my * H, H, axis=2)
    V = lax.dynamic_slice_in_dim(V_ext, my * H, H, axis=2)
    Kt = jnp.transpose(K, (0, 2, 3, 1)).astype(bf)
    Vt = jnp.transpose(V, (0, 2, 1, 3)).astype(bf)

    m = B * SQ
    half_buf = pltpu.VMEM((m, HALF), bf)
    out = pl.pallas_call(
        _fused_body,
        out_shape=jax.ShapeDtypeStruct((m, DM), jnp.bfloat16),
        in_specs=[pl.BlockSpec(memory_space=pltpu.VMEM)] * 5,
        out_specs=pl.BlockSpec(memory_space=pltpu.VMEM),
        scratch_shapes=[
            half_buf, half_buf,
            half_buf, half_buf,
            half_buf, half_buf, half_buf, half_buf,
            pltpu.SemaphoreType.DMA((4 * NCHUNK,)),
            pltpu.SemaphoreType.DMA((4 * NCHUNK,)),
        ],
        compiler_params=pltpu.CompilerParams(collective_id=0),
    )(x, Wq, Kt, Vt, Wo)
    return out.reshape(B, SQ, DM)
